# baseline (speedup 1.0000x reference)
import functools

import jax
import jax.numpy as jnp
from jax import lax
from jax.experimental import pallas as pl
from jax.experimental.pallas import tpu as pltpu
from jax.experimental.pallas import tpu_sc as plsc

_NC = 2
_NS = 16
_NW = _NC * _NS
_GRP = 10


def _make_gather(batch, timesteps, embed_dim):
  total = batch * timesteps
  n_units = total // 128
  u_per_w = n_units // _NW
  n_groups = u_per_w // _GRP
  assert n_groups % 2 == 0
  n_eg = embed_dim // 8
  n_bg = batch // 128
  mesh = plsc.VectorSubcoreMesh(core_axis_name="c", subcore_axis_name="s")

  @functools.partial(
      pl.kernel,
      mesh=mesh,
      out_type=jax.ShapeDtypeStruct((timesteps, n_eg, n_bg, 8, 128),
                                    jnp.float32),
      scratch_types=[
          pltpu.VMEM((n_groups, _GRP * 128), jnp.int32),
          pltpu.VMEM((_GRP * 128, embed_dim), jnp.float32),
          pltpu.VMEM((_GRP * 128, embed_dim), jnp.float32),
          pltpu.VMEM((n_eg, 8, 128), jnp.float32),
          pltpu.VMEM((n_eg, 8, 128), jnp.float32),
          pltpu.SemaphoreType.DMA,
          pltpu.SemaphoreType.DMA,
          pltpu.SemaphoreType.DMA,
          pltpu.SemaphoreType.DMA,
      ],
      compiler_params=pltpu.CompilerParams(
          use_tc_tiling_on_sc=False, needs_layout_passes=False),
  )
  def gather_kernel(idx_hbm, table_hbm, out_hbm, idx_v, rows_0, rows_1,
                    tbuf_0, tbuf_1, sem_g0, sem_g1, sem_w0, sem_w1):
    wid = lax.axis_index("s") * _NC + lax.axis_index("c")
    u_base = wid * u_per_w
    rows = (rows_0, rows_1)
    tbuf = (tbuf_0, tbuf_1)
    sem_g = (sem_g0, sem_g1)
    sem_w = (sem_w0, sem_w1)

    pltpu.sync_copy(idx_hbm.at[wid], idx_v)

    def gather_start(j, b):
      pltpu.async_copy(table_hbm.at[idx_v.at[j]], rows[b], sem_g[b])

    def gather_wait(j, b):
      pltpu.make_async_copy(table_hbm.at[idx_v.at[j]], rows[b],
                            sem_g[b]).wait()

    def tile_pos(u):
      return u // n_bg, u % n_bg

    def write_start(u, p):
      t, bg = tile_pos(u)
      for eg in range(n_eg):
        pltpu.async_copy(tbuf[p].at[eg], out_hbm.at[t, eg, bg], sem_w[p])

    def write_wait(u, p):
      t, bg = tile_pos(u)
      for eg in range(n_eg):
        pltpu.make_async_copy(tbuf[p].at[eg], out_hbm.at[t, eg, bg],
                              sem_w[p]).wait()

    def unit(j, kk, p, b):
      u = u_base + j * _GRP + kk
      lane = lax.iota(jnp.int32, 16)

      @pl.when(j * _GRP + kk >= 2)
      def _():
        write_wait(u - 2, p)

      r0 = kk * 128
      for eg in range(n_eg):
        for er in range(8):
          col = jnp.full((16,), eg * 8 + er, jnp.int32)
          for jb in range(8):
            row_ids = r0 + jb * 16 + lane
            vals = plsc.load_gather(rows[b], [row_ids, col])
            tbuf[p][eg, er, pl.ds(jb * 16, 16)] = vals
      write_start(u, p)

    def step(j, b):
      @pl.when(j + 1 < n_groups)
      def _():
        gather_start(j + 1, 1 - b)

      gather_wait(j, b)

      def pair(kp, carry):
        unit(j, 2 * kp, 0, b)
        unit(j, 2 * kp + 1, 1, b)
        return carry

      lax.fori_loop(0, _GRP // 2, pair, 0)

    gather_start(0, 0)

    def body(jj, carry):
      step(2 * jj, 0)
      step(2 * jj + 1, 1)
      return carry

    lax.fori_loop(0, n_groups // 2, body, 0)
    write_wait(u_base + u_per_w - 2, 0)
    write_wait(u_base + u_per_w - 1, 1)

  return gather_kernel


def kernel(x, table):
  batch, timesteps = x.shape
  vocab, embed_dim = table.shape
  total = batch * timesteps
  assert batch % 128 == 0 and embed_dim % 8 == 0
  assert total % (_NW * _GRP * 128) == 0
  n_groups = total // (_NW * _GRP * 128)
  idx = x.T.reshape(_NW, n_groups, _GRP * 128).astype(jnp.int32)
  out5 = _make_gather(batch, timesteps, embed_dim)(idx, table)
  return out5.transpose(2, 4, 0, 1, 3).reshape(batch, timesteps, embed_dim)


# speedup vs baseline: 1.6168x; 1.6168x over previous
import functools

import jax
import jax.numpy as jnp
from jax import lax
from jax.experimental import pallas as pl
from jax.experimental.pallas import tpu as pltpu
from jax.experimental.pallas import tpu_sc as plsc

_NC = 2
_NS = 16
_NW = _NC * _NS
_GRP = 4


def _make_gather(batch, timesteps, embed_dim):
  total = batch * timesteps
  n_units = total // 128
  u_per_w = n_units // _NW
  n_groups = u_per_w // _GRP
  assert n_groups % 2 == 0
  n_eg = embed_dim // 8
  n_bg = batch // 128
  mesh = plsc.VectorSubcoreMesh(core_axis_name="c", subcore_axis_name="s")

  @functools.partial(
      pl.kernel,
      mesh=mesh,
      out_type=jax.ShapeDtypeStruct((timesteps, n_eg, n_bg, 8, 128),
                                    jnp.float32),
      scratch_types=[
          pltpu.VMEM((n_groups, _GRP * 128), jnp.int32),
          pltpu.VMEM((_GRP * 128, embed_dim), jnp.float32),
          pltpu.VMEM((_GRP * 128, embed_dim), jnp.float32),
          pltpu.VMEM((embed_dim, 128), jnp.float32),
          pltpu.VMEM((embed_dim, 128), jnp.float32),
          pltpu.SemaphoreType.DMA,
          pltpu.SemaphoreType.DMA,
          pltpu.SemaphoreType.DMA,
          pltpu.SemaphoreType.DMA,
      ],
      compiler_params=pltpu.CompilerParams(
          use_tc_tiling_on_sc=False, needs_layout_passes=False),
  )
  def gather_kernel(idx_hbm, table_hbm, out_hbm, idx_v, rows_0, rows_1,
                    tbuf_0, tbuf_1, sem_g0, sem_g1, sem_w0, sem_w1):
    wid = lax.axis_index("s") * _NC + lax.axis_index("c")
    u_base = wid * u_per_w
    rows = (rows_0, rows_1)
    tbuf = (tbuf_0, tbuf_1)
    sem_g = (sem_g0, sem_g1)
    sem_w = (sem_w0, sem_w1)

    pltpu.sync_copy(idx_hbm.at[wid], idx_v)

    def gather_start(j, b):
      pltpu.async_copy(table_hbm.at[idx_v.at[j]], rows[b], sem_g[b])

    def gather_wait(j, b):
      pltpu.make_async_copy(table_hbm.at[idx_v.at[j]], rows[b],
                            sem_g[b]).wait()

    def tile_pos(u):
      return u // n_bg, u % n_bg

    def write_start(u, p):
      t, bg = tile_pos(u)
      for eg in range(n_eg):
        pltpu.async_copy(tbuf[p].at[pl.ds(eg * 8, 8)], out_hbm.at[t, eg, bg],
                         sem_w[p])

    def write_wait(u, p):
      t, bg = tile_pos(u)
      for eg in range(n_eg):
        pltpu.make_async_copy(tbuf[p].at[pl.ds(eg * 8, 8)],
                              out_hbm.at[t, eg, bg], sem_w[p]).wait()

    def unit(j, kk, p, b):
      u = u_base + j * _GRP + kk
      lane = lax.iota(jnp.int32, 16)
      wrapped = [(lane + d) & 15 for d in range(16)]

      @pl.when(j * _GRP + kk >= 2)
      def _():
        write_wait(u - 2, p)

      r0 = kk * 128

      def iblock(ib, carry):
        row_ids = r0 + ib * 16 + lane
        dst_col = ib * 16 + lane
        for cb in range(embed_dim // 16):
          for d in range(16):
            col = cb * 16 + wrapped[d]
            vals = plsc.load_gather(rows[b], [row_ids, col])
            plsc.store_scatter(tbuf[p], [col, dst_col], vals)
        return carry

      lax.fori_loop(0, 8, iblock, 0)
      write_start(u, p)

    def step(j, b):
      @pl.when(j + 1 < n_groups)
      def _():
        gather_start(j + 1, 1 - b)

      gather_wait(j, b)

      def pair(kp, carry):
        unit(j, 2 * kp, 0, b)
        unit(j, 2 * kp + 1, 1, b)
        return carry

      lax.fori_loop(0, _GRP // 2, pair, 0)

    gather_start(0, 0)

    def body(jj, carry):
      step(2 * jj, 0)
      step(2 * jj + 1, 1)
      return carry

    lax.fori_loop(0, n_groups // 2, body, 0)
    write_wait(u_base + u_per_w - 2, 0)
    write_wait(u_base + u_per_w - 1, 1)

  return gather_kernel


def kernel(x, table):
  batch, timesteps = x.shape
  vocab, embed_dim = table.shape
  total = batch * timesteps
  assert batch % 128 == 0 and embed_dim % 8 == 0
  assert total % (_NW * _GRP * 128) == 0
  n_groups = total // (_NW * _GRP * 128)
  idx = x.T.reshape(_NW, n_groups, _GRP * 128).astype(jnp.int32)
  out5 = _make_gather(batch, timesteps, embed_dim)(idx, table)
  return out5.transpose(2, 4, 0, 1, 3).reshape(batch, timesteps, embed_dim)


# speedup vs baseline: 1.6178x; 1.0006x over previous
import functools

import jax
import jax.numpy as jnp
from jax import lax
from jax.experimental import pallas as pl
from jax.experimental.pallas import tpu as pltpu
from jax.experimental.pallas import tpu_sc as plsc

_NC = 2
_NS = 16
_NW = _NC * _NS
_GRP = 4


def _make_gather(batch, timesteps, embed_dim):
  total = batch * timesteps
  n_units = total // 128
  u_per_w = n_units // _NW
  n_groups = u_per_w // _GRP
  assert n_groups % 2 == 0
  n_eg = embed_dim // 8
  n_bg = batch // 128
  mesh = plsc.VectorSubcoreMesh(core_axis_name="c", subcore_axis_name="s")

  @functools.partial(
      pl.kernel,
      mesh=mesh,
      out_type=jax.ShapeDtypeStruct((timesteps, n_eg, n_bg, 8, 128),
                                    jnp.float32),
      scratch_types=[
          pltpu.VMEM((n_groups, _GRP * 128), jnp.int32),
          pltpu.VMEM((_GRP * 128, embed_dim), jnp.float32),
          pltpu.VMEM((_GRP * 128, embed_dim), jnp.float32),
          pltpu.VMEM((embed_dim, 128), jnp.float32),
          pltpu.VMEM((embed_dim, 128), jnp.float32),
          pltpu.SemaphoreType.DMA,
          pltpu.SemaphoreType.DMA,
          pltpu.SemaphoreType.DMA,
          pltpu.SemaphoreType.DMA,
      ],
      compiler_params=pltpu.CompilerParams(
          use_tc_tiling_on_sc=False, needs_layout_passes=False),
  )
  def gather_kernel(idx_hbm, table_hbm, out_hbm, idx_v, rows_0, rows_1,
                    tbuf_0, tbuf_1, sem_g0, sem_g1, sem_w0, sem_w1):
    wid = lax.axis_index("s") * _NC + lax.axis_index("c")
    u_base = wid * u_per_w
    rows = (rows_0, rows_1)
    tbuf = (tbuf_0, tbuf_1)
    sem_g = (sem_g0, sem_g1)
    sem_w = (sem_w0, sem_w1)

    pltpu.sync_copy(idx_hbm.at[wid], idx_v)

    def gather_start(j, b):
      pltpu.async_copy(table_hbm.at[idx_v.at[j]], rows[b], sem_g[b])

    def gather_wait(j, b):
      pltpu.make_async_copy(table_hbm.at[idx_v.at[j]], rows[b],
                            sem_g[b]).wait()

    def tile_pos(u):
      return u // n_bg, u % n_bg

    def write_start(u, p):
      t, bg = tile_pos(u)
      for eg in range(n_eg):
        pltpu.async_copy(tbuf[p].at[pl.ds(eg * 8, 8)], out_hbm.at[t, eg, bg],
                         sem_w[p])

    def write_wait(u, p):
      t, bg = tile_pos(u)
      for eg in range(n_eg):
        pltpu.make_async_copy(tbuf[p].at[pl.ds(eg * 8, 8)],
                              out_hbm.at[t, eg, bg], sem_w[p]).wait()

    def unit(j, kk, p, b):
      u = u_base + j * _GRP + kk
      lane = lax.iota(jnp.int32, 16)
      cols = [cb * 16 + ((lane + d) & 15)
              for cb in range(embed_dim // 16) for d in range(16)]

      @pl.when(j * _GRP + kk >= 2)
      def _():
        write_wait(u - 2, p)

      r0 = kk * 128

      def iblock(ib, carry):
        row_ids = r0 + ib * 16 + lane
        dst_col = ib * 16 + lane
        for col in cols:
          vals = plsc.load_gather(rows[b], [row_ids, col])
          plsc.store_scatter(tbuf[p], [col, dst_col], vals)
        return carry

      lax.fori_loop(0, 8, iblock, 0)
      write_start(u, p)

    def step(j, b):
      @pl.when(j + 1 < n_groups)
      def _():
        gather_start(j + 1, 1 - b)

      gather_wait(j, b)

      def pair(kp, carry):
        unit(j, 2 * kp, 0, b)
        unit(j, 2 * kp + 1, 1, b)
        return carry

      lax.fori_loop(0, _GRP // 2, pair, 0)

    gather_start(0, 0)

    def body(jj, carry):
      step(2 * jj, 0)
      step(2 * jj + 1, 1)
      return carry

    lax.fori_loop(0, n_groups // 2, body, 0)
    write_wait(u_base + u_per_w - 2, 0)
    write_wait(u_base + u_per_w - 1, 1)

  return gather_kernel


def kernel(x, table):
  batch, timesteps = x.shape
  vocab, embed_dim = table.shape
  total = batch * timesteps
  assert batch % 128 == 0 and embed_dim % 8 == 0
  assert total % (_NW * _GRP * 128) == 0
  n_groups = total // (_NW * _GRP * 128)
  idx = x.T.reshape(_NW, n_groups, _GRP * 128).astype(jnp.int32)
  out5 = _make_gather(batch, timesteps, embed_dim)(idx, table)
  return out5.transpose(2, 4, 0, 1, 3).reshape(batch, timesteps, embed_dim)
